# baseline (speedup 1.0000x reference)
import jax
import jax.numpy as jnp
from jax import lax
from jax.experimental import pallas as pl
from jax.experimental.pallas import tpu as pltpu
from jax.experimental.pallas import tpu_sc as plsc

_EMB = 32
_NC = 2
_NS = 16
_NW = _NC * _NS

_B = 16384 * 26
_BPW = _B // _NW
_C = 1664
_NCHUNK = _BPW // _C


def _emb_body(x_hbm, w_hbm, out_hbm,
              idx0, idx1, rows0, rows1,
              si0, si1, sg0, sg1, ss0, ss1):
    idx = (idx0, idx1)
    rows = (rows0, rows1)
    si = (si0, si1)
    sg = (sg0, sg1)
    ss = (ss0, ss1)
    wid = lax.axis_index("s") * _NC + lax.axis_index("c")
    wbase = wid * _BPW

    def idx_copy(i):
        b = i % 2
        return pltpu.make_async_copy(
            x_hbm.at[pl.ds(wbase + i * _C, _C)], idx[b], si[b])

    def gather_copy(i):
        b = i % 2
        return pltpu.make_async_copy(w_hbm.at[idx[b]], rows[b], sg[b])

    def store_copy(i):
        b = i % 2
        return pltpu.make_async_copy(
            rows[b], out_hbm.at[pl.ds(wbase + i * _C, _C)], ss[b])

    idx_copy(0).start()
    idx_copy(1).start()
    for i in range(_NCHUNK):
        idx_copy(i).wait()
        if i >= 2:
            store_copy(i - 2).wait()
        gather_copy(i).start()
        gather_copy(i).wait()
        store_copy(i).start()
        if i + 2 < _NCHUNK:
            idx_copy(i + 2).start()
    store_copy(_NCHUNK - 2).wait()
    store_copy(_NCHUNK - 1).wait()


def kernel(x, weight):
    bb, ff = x.shape
    xf = x.reshape(bb * ff).astype(jnp.int32)
    run = pl.kernel(
        _emb_body,
        out_type=jax.ShapeDtypeStruct((bb * ff, _EMB), jnp.float32),
        mesh=plsc.VectorSubcoreMesh(core_axis_name="c", subcore_axis_name="s"),
        compiler_params=pltpu.CompilerParams(use_tc_tiling_on_sc=False),
        scratch_types=[
            pltpu.VMEM((_C,), jnp.int32),
            pltpu.VMEM((_C,), jnp.int32),
            pltpu.VMEM((_C, _EMB), jnp.float32),
            pltpu.VMEM((_C, _EMB), jnp.float32),
            pltpu.SemaphoreType.DMA,
            pltpu.SemaphoreType.DMA,
            pltpu.SemaphoreType.DMA,
            pltpu.SemaphoreType.DMA,
            pltpu.SemaphoreType.DMA,
            pltpu.SemaphoreType.DMA,
        ],
    )
    out = run(xf, weight)
    return out.reshape(bb, ff, _EMB)


# speedup vs baseline: 1.0001x; 1.0001x over previous
import jax
import jax.numpy as jnp
from jax import lax
from jax.experimental import pallas as pl
from jax.experimental.pallas import tpu as pltpu
from jax.experimental.pallas import tpu_sc as plsc

_EMB = 32
_NC = 2
_NS = 16
_NW = _NC * _NS

_B = 16384 * 26
_BPW = _B // _NW
_C = 1664
_NCHUNK = _BPW // _C


def _emb_body(x_hbm, w_hbm, out_hbm,
              idx0, idx1, rows0, rows1,
              si0, si1, sg0, sg1, ss0, ss1):
    idx = (idx0, idx1)
    rows = (rows0, rows1)
    si = (si0, si1)
    sg = (sg0, sg1)
    ss = (ss0, ss1)
    wid = lax.axis_index("s") * _NC + lax.axis_index("c")
    wbase = wid * _BPW

    def idx_copy(i):
        b = i % 2
        return pltpu.make_async_copy(
            x_hbm.at[pl.ds(wbase + i * _C, _C)], idx[b], si[b])

    def gather_copy(i):
        b = i % 2
        return pltpu.make_async_copy(w_hbm.at[idx[b]], rows[b], sg[b])

    def store_copy(i):
        b = i % 2
        return pltpu.make_async_copy(
            rows[b], out_hbm.at[pl.ds(wbase + i * _C, _C)], ss[b])

    idx_copy(0).start()
    idx_copy(1).start()
    for i in range(_NCHUNK):
        idx_copy(i).wait()
        if i >= 2:
            store_copy(i - 2).wait()
        gather_copy(i).start()
        gather_copy(i).wait()
        store_copy(i).start()
        if i + 2 < _NCHUNK:
            idx_copy(i + 2).start()
    store_copy(_NCHUNK - 2).wait()
    store_copy(_NCHUNK - 1).wait()


def kernel(x, weight):
    bb, ff = x.shape
    xf = x.reshape(bb * ff).astype(jnp.int32)
    w4 = weight.reshape(weight.shape[0] // 4, 4 * _EMB)
    w4 = lax.optimization_barrier(w4)
    wl = w4.reshape(weight.shape[0], _EMB)
    run = pl.kernel(
        _emb_body,
        out_type=jax.ShapeDtypeStruct((bb * ff, _EMB), jnp.float32),
        mesh=plsc.VectorSubcoreMesh(core_axis_name="c", subcore_axis_name="s"),
        compiler_params=pltpu.CompilerParams(use_tc_tiling_on_sc=False),
        scratch_types=[
            pltpu.VMEM((_C,), jnp.int32),
            pltpu.VMEM((_C,), jnp.int32),
            pltpu.VMEM((_C, _EMB), jnp.float32),
            pltpu.VMEM((_C, _EMB), jnp.float32),
            pltpu.SemaphoreType.DMA,
            pltpu.SemaphoreType.DMA,
            pltpu.SemaphoreType.DMA,
            pltpu.SemaphoreType.DMA,
            pltpu.SemaphoreType.DMA,
            pltpu.SemaphoreType.DMA,
        ],
    )
    out = run(xf, wl)
    return out.reshape(bb, ff, _EMB)


# speedup vs baseline: 1.1462x; 1.1461x over previous
import jax
import jax.numpy as jnp
from jax import lax
from jax.experimental import pallas as pl
from jax.experimental.pallas import tpu as pltpu
from jax.experimental.pallas import tpu_sc as plsc

_EMB = 32
_NC = 2
_NS = 16
_NW = _NC * _NS

_B = 16384 * 26
_BPW = _B // _NW
_C = 1664
_NCHUNK = _BPW // _C

_V = 1000000
_RCOLS = 4096
_RGRID = -(-_V // _RCOLS)


def _repack_body(wt_ref, o_ref):
    t3 = wt_ref[...].T.reshape(_RCOLS // 4, 4, _EMB)
    for q in range(4):
        o_ref[:, q * _EMB:(q + 1) * _EMB] = t3[:, q, :]


def _emb_body(x_hbm, w_hbm, out_hbm,
              idx0, idx1, rows0, rows1,
              si0, si1, sg0, sg1, ss0, ss1):
    idx = (idx0, idx1)
    rows = (rows0, rows1)
    si = (si0, si1)
    sg = (sg0, sg1)
    ss = (ss0, ss1)
    wid = lax.axis_index("s") * _NC + lax.axis_index("c")
    wbase = wid * _BPW

    def idx_copy(i):
        b = i % 2
        return pltpu.make_async_copy(
            x_hbm.at[pl.ds(wbase + i * _C, _C)], idx[b], si[b])

    def gather_copy(i):
        b = i % 2
        return pltpu.make_async_copy(w_hbm.at[idx[b]], rows[b], sg[b])

    def store_copy(i):
        b = i % 2
        return pltpu.make_async_copy(
            rows[b], out_hbm.at[pl.ds(wbase + i * _C, _C)], ss[b])

    idx_copy(0).start()
    idx_copy(1).start()
    for i in range(_NCHUNK):
        idx_copy(i).wait()
        if i >= 2:
            store_copy(i - 2).wait()
        gather_copy(i).start()
        gather_copy(i).wait()
        store_copy(i).start()
        if i + 2 < _NCHUNK:
            idx_copy(i + 2).start()
    store_copy(_NCHUNK - 2).wait()
    store_copy(_NCHUNK - 1).wait()


def kernel(x, weight):
    bb, ff = x.shape
    xf = x.reshape(bb * ff).astype(jnp.int32)

    wt = weight.T
    wl4 = pl.pallas_call(
        _repack_body,
        grid=(_RGRID,),
        in_specs=[pl.BlockSpec((_EMB, _RCOLS), lambda i: (0, i))],
        out_specs=pl.BlockSpec((_RCOLS // 4, 4 * _EMB), lambda i: (i, 0)),
        out_shape=jax.ShapeDtypeStruct((_V // 4, 4 * _EMB), jnp.float32),
    )(wt)
    wl4 = lax.optimization_barrier(wl4)
    wl = wl4.reshape(_V, _EMB)

    run = pl.kernel(
        _emb_body,
        out_type=jax.ShapeDtypeStruct((bb * ff, _EMB), jnp.float32),
        mesh=plsc.VectorSubcoreMesh(core_axis_name="c", subcore_axis_name="s"),
        compiler_params=pltpu.CompilerParams(use_tc_tiling_on_sc=False),
        scratch_types=[
            pltpu.VMEM((_C,), jnp.int32),
            pltpu.VMEM((_C,), jnp.int32),
            pltpu.VMEM((_C, _EMB), jnp.float32),
            pltpu.VMEM((_C, _EMB), jnp.float32),
            pltpu.SemaphoreType.DMA,
            pltpu.SemaphoreType.DMA,
            pltpu.SemaphoreType.DMA,
            pltpu.SemaphoreType.DMA,
            pltpu.SemaphoreType.DMA,
            pltpu.SemaphoreType.DMA,
        ],
    )
    out = run(xf, wl)
    return out.reshape(bb, ff, _EMB)


# speedup vs baseline: 1.2025x; 1.0491x over previous
import jax
import jax.numpy as jnp
from jax import lax
from jax.experimental import pallas as pl
from jax.experimental.pallas import tpu as pltpu
from jax.experimental.pallas import tpu_sc as plsc

_EMB = 32
_NC = 2
_NS = 16
_NW = _NC * _NS

_B = 16384 * 26
_BPW = _B // _NW
_C = 1664
_NCHUNK = _BPW // _C

_V = 1000000
_RCOLS = 16384
_RGRID = -(-_V // _RCOLS)


def _repack_body(wt_ref, o_ref):
    t3 = wt_ref[...].T.reshape(_RCOLS // 4, 4, _EMB)
    for q in range(4):
        o_ref[:, q * _EMB:(q + 1) * _EMB] = t3[:, q, :]


def _emb_body(x_hbm, w_hbm, out_hbm,
              idx0, idx1, rows0, rows1,
              si0, si1, sg0, sg1, ss0, ss1):
    idx = (idx0, idx1)
    rows = (rows0, rows1)
    si = (si0, si1)
    sg = (sg0, sg1)
    ss = (ss0, ss1)
    wid = lax.axis_index("s") * _NC + lax.axis_index("c")
    wbase = wid * _BPW

    def idx_copy(i):
        b = i % 2
        return pltpu.make_async_copy(
            x_hbm.at[pl.ds(wbase + i * _C, _C)], idx[b], si[b])

    def gather_copy(i):
        b = i % 2
        return pltpu.make_async_copy(w_hbm.at[idx[b]], rows[b], sg[b])

    def store_copy(i):
        b = i % 2
        return pltpu.make_async_copy(
            rows[b], out_hbm.at[pl.ds(wbase + i * _C, _C)], ss[b])

    idx_copy(0).start()
    idx_copy(1).start()
    for i in range(_NCHUNK):
        idx_copy(i).wait()
        if i >= 2:
            store_copy(i - 2).wait()
        gather_copy(i).start()
        gather_copy(i).wait()
        store_copy(i).start()
        if i + 2 < _NCHUNK:
            idx_copy(i + 2).start()
    store_copy(_NCHUNK - 2).wait()
    store_copy(_NCHUNK - 1).wait()


def kernel(x, weight):
    bb, ff = x.shape
    xf = x.reshape(bb * ff).astype(jnp.int32)

    wt = weight.T
    wl4 = pl.pallas_call(
        _repack_body,
        grid=(_RGRID,),
        in_specs=[pl.BlockSpec((_EMB, _RCOLS), lambda i: (0, i))],
        out_specs=pl.BlockSpec((_RCOLS // 4, 4 * _EMB), lambda i: (i, 0)),
        out_shape=jax.ShapeDtypeStruct((_V // 4, 4 * _EMB), jnp.float32),
    )(wt)
    wl4 = lax.optimization_barrier(wl4)
    wl = wl4.reshape(_V, _EMB)

    run = pl.kernel(
        _emb_body,
        out_type=jax.ShapeDtypeStruct((bb * ff, _EMB), jnp.float32),
        mesh=plsc.VectorSubcoreMesh(core_axis_name="c", subcore_axis_name="s"),
        compiler_params=pltpu.CompilerParams(use_tc_tiling_on_sc=False),
        scratch_types=[
            pltpu.VMEM((_C,), jnp.int32),
            pltpu.VMEM((_C,), jnp.int32),
            pltpu.VMEM((_C, _EMB), jnp.float32),
            pltpu.VMEM((_C, _EMB), jnp.float32),
            pltpu.SemaphoreType.DMA,
            pltpu.SemaphoreType.DMA,
            pltpu.SemaphoreType.DMA,
            pltpu.SemaphoreType.DMA,
            pltpu.SemaphoreType.DMA,
            pltpu.SemaphoreType.DMA,
        ],
    )
    out = run(xf, wl)
    return out.reshape(bb, ff, _EMB)
